# baseline (speedup 1.0000x reference)
import functools

import jax
import jax.numpy as jnp
from jax import lax
from jax.experimental import pallas as pl
from jax.experimental.pallas import tpu as pltpu
from jax.experimental.pallas import tpu_sc as plsc

N = 10000
D = 128
DP = 144
E = 320000
NC, NS = 2, 16
NW = NC * NS
EPW = 10240
ECH = 80
AGG_ROWS = 10112
ZSTRIPE = 632

KNN_CB = 2000
KNN_NCH = 5
KNN_QB = 400

GIDX = 32768


def _build_sc_scatter():
    mesh = plsc.VectorSubcoreMesh(core_axis_name="c", subcore_axis_name="s")

    @functools.partial(
        pl.kernel,
        out_type=jax.ShapeDtypeStruct((NC, N, DP), jnp.float32),
        mesh=mesh,
        scratch_types=[
            pltpu.VMEM((2, 128), jnp.int32),
            pltpu.VMEM((2, 128), jnp.int32),
            pltpu.VMEM((2, 128, DP), jnp.float32),
            pltpu.VMEM_SHARED((AGG_ROWS, DP), jnp.float32),
            pltpu.SemaphoreType.DMA,
            pltpu.SemaphoreType.DMA,
            pltpu.SemaphoreType.DMA,
            pltpu.SemaphoreType.DMA,
        ],
        compiler_params=pltpu.CompilerParams(use_tc_tiling_on_sc=False),
    )
    def sc_scatter(table, src_r, dst_r, zeros, out, src_v, dst_v, rows_v,
                   agg_sh, sem_i, sem_g0, sem_g1, sem_s):
        c = lax.axis_index("c")
        s = lax.axis_index("s")
        pltpu.sync_copy(zeros, agg_sh.at[pl.ds(s * ZSTRIPE, ZSTRIPE)])
        plsc.subcore_barrier()
        wid = c * NS + s

        sem_g = [sem_g0, sem_g1]

        @pl.loop(0, ECH, step=2)
        def _group(j):
            ids = []
            for b in range(2):
                ids.append(pltpu.async_copy(src_r.at[wid, j + b],
                                            src_v.at[b], sem_i))
                ids.append(pltpu.async_copy(dst_r.at[wid, j + b],
                                            dst_v.at[b], sem_i))
            for d in ids:
                d.wait()
            gds = [pltpu.async_copy(table.at[src_v.at[b]],
                                    rows_v.at[b], sem_g[b]) for b in range(2)]
            sds = []
            for b in range(2):
                gds[b].wait()
                sds.append(pltpu.async_copy(rows_v.at[b],
                                            agg_sh.at[dst_v.at[b]],
                                            sem_s, add=True))
            for d in sds:
                d.wait()

        plsc.subcore_barrier()

        @pl.when(s < NS - 1)
        def _copy_full():
            pltpu.sync_copy(agg_sh.at[pl.ds(s * ZSTRIPE, ZSTRIPE)],
                            out.at[c, pl.ds(s * ZSTRIPE, ZSTRIPE)])

        @pl.when(s == NS - 1)
        def _copy_tail():
            pltpu.sync_copy(agg_sh.at[pl.ds((NS - 1) * ZSTRIPE, N - (NS - 1) * ZSTRIPE)],
                            out.at[c, pl.ds((NS - 1) * ZSTRIPE, N - (NS - 1) * ZSTRIPE)])

    return sc_scatter


def _build_sc_gather():
    mesh = plsc.VectorSubcoreMesh(core_axis_name="c", subcore_axis_name="s")

    @functools.partial(
        pl.kernel,
        out_type=jax.ShapeDtypeStruct((GIDX, DP), jnp.float32),
        mesh=mesh,
        scratch_types=[
            pltpu.VMEM((8, 128), jnp.int32),
            pltpu.VMEM((2, 128, DP), jnp.float32),
            pltpu.SemaphoreType.DMA,
            pltpu.SemaphoreType.DMA,
            pltpu.SemaphoreType.DMA,
        ],
        compiler_params=pltpu.CompilerParams(use_tc_tiling_on_sc=False),
    )
    def sc_gather(table, idx_r, out, idx_v, rows_v, sem_g0, sem_g1, sem_s):
        c = lax.axis_index("c")
        s = lax.axis_index("s")
        wid = c * NS + s
        pltpu.sync_copy(idx_r.at[wid], idx_v)
        base = wid * 1024

        sem_g = [sem_g0, sem_g1]

        @pl.loop(0, 8, step=2)
        def _group(j):
            gds = [pltpu.async_copy(table.at[idx_v.at[j + b]], rows_v.at[b],
                                    sem_g[b]) for b in range(2)]
            sds = []
            for b in range(2):
                gds[b].wait()
                sds.append(pltpu.async_copy(
                    rows_v.at[b], out.at[pl.ds(base + (j + b) * 128, 128)],
                    sem_s))
            for d in sds:
                d.wait()

    return sc_gather


def _enc_kernel(x_ref, w1_ref, b1_ref, w2_ref, b2_ref, out_ref):
    x = x_ref[...]
    h = lax.dot_general(x, w1_ref[...], (((1,), (1,)), ((), ())),
                        preferred_element_type=jnp.float32) + b1_ref[...]
    h = jnp.maximum(h, 0.0)
    y = lax.dot_general(h, w2_ref[...], (((1,), (1,)), ((), ())),
                        preferred_element_type=jnp.float32) + b2_ref[...]
    out_ref[:, :D] = y
    out_ref[:, D:] = jnp.ones((out_ref.shape[0], DP - D), jnp.float32)


def _encoder(xin, W1, b1, W2, b2):
    BM = 1000
    nb = N // BM
    return pl.pallas_call(
        _enc_kernel,
        grid=(nb,),
        in_specs=[
            pl.BlockSpec((BM, xin.shape[1]), lambda i: (i, 0)),
            pl.BlockSpec(W1.shape, lambda i: (0, 0)),
            pl.BlockSpec((1, D), lambda i: (0, 0)),
            pl.BlockSpec(W2.shape, lambda i: (0, 0)),
            pl.BlockSpec((1, D), lambda i: (0, 0)),
        ],
        out_specs=pl.BlockSpec((BM, DP), lambda i: (i, 0)),
        out_shape=jax.ShapeDtypeStruct((N, DP), jnp.float32),
    )(xin, W1, b1.reshape(1, D), W2, b2.reshape(1, D))


def _sage_dense_kernel(p_ref, x_ref, wl_ref, wr_ref, out_ref,
                       acc_agg, acc_y, yraw, *, bm):
    ph = pl.program_id(0)
    i = pl.program_id(1)

    @pl.when(jnp.logical_and(ph == 0, i == 0))
    def _init():
        acc_agg[...] = jnp.zeros_like(acc_agg)
        acc_y[...] = jnp.zeros_like(acc_y)

    def _agg_block():
        feat = p_ref[0, :, :D] + p_ref[1, :, :D]
        cnt = p_ref[0, :, D:D + 1] + p_ref[1, :, D:D + 1]
        return feat / jnp.clip(cnt, 1.0, None)

    @pl.when(ph == 0)
    def _phase0():
        acc_agg[...] += jnp.sum(_agg_block(), axis=0, keepdims=True)

    @pl.when(ph == 1)
    def _phase1():
        agg = _agg_block() - acc_agg[...] / float(N)
        xf = x_ref[:, :D]
        t = lax.dot_general(agg, wl_ref[...], (((1,), (1,)), ((), ())),
                            preferred_element_type=jnp.float32)
        t = t + lax.dot_general(xf, wr_ref[...], (((1,), (1,)), ((), ())),
                                preferred_element_type=jnp.float32)
        yr = xf + jnp.maximum(t, 0.0)
        acc_y[...] += jnp.sum(yr, axis=0, keepdims=True)
        yraw[pl.ds(i * bm, bm), :] = yr

    @pl.when(ph == 2)
    def _phase2():
        out_ref[:, :D] = yraw[pl.ds(i * bm, bm), :] - acc_y[...] / float(N)
        out_ref[:, D:] = jnp.ones((bm, DP - D), jnp.float32)


def _sage_dense(p, x, Wl, Wr):
    BM = 1000
    nb = N // BM
    return pl.pallas_call(
        functools.partial(_sage_dense_kernel, bm=BM),
        grid=(3, nb),
        in_specs=[
            pl.BlockSpec((2, BM, DP), lambda ph, i: (0, i, 0)),
            pl.BlockSpec((BM, DP), lambda ph, i: (i, 0)),
            pl.BlockSpec((D, D), lambda ph, i: (0, 0)),
            pl.BlockSpec((D, D), lambda ph, i: (0, 0)),
        ],
        out_specs=pl.BlockSpec((BM, DP), lambda ph, i: (i, 0)),
        out_shape=jax.ShapeDtypeStruct((N, DP), jnp.float32),
        scratch_shapes=[
            pltpu.VMEM((1, D), jnp.float32),
            pltpu.VMEM((1, D), jnp.float32),
            pltpu.VMEM((N, D), jnp.float32),
        ],
    )(p, x, Wl, Wr)


def _knn_kernel(q_ref, c_ref, sq_ref, idx_ref, vals, inds):
    ch = pl.program_id(1)

    @pl.when(ch == 0)
    def _init():
        vals[...] = jnp.full_like(vals[...], jnp.inf)
        inds[...] = jnp.zeros_like(inds[...])

    q = q_ref[...]
    c = c_ref[0]
    d2 = (q[:, 0:1] - c[0:1, :]) ** 2
    d2 = d2 + (q[:, 1:2] - c[1:2, :]) ** 2
    d2 = d2 + (q[:, 2:3] - c[2:3, :]) ** 2

    base = ch * KNN_CB
    lane = lax.broadcasted_iota(jnp.int32, d2.shape, 1)
    cvs, cis = [], []
    work = d2
    for _ in range(3):
        m = jnp.min(work, axis=1, keepdims=True)
        am = jnp.argmin(work, axis=1).astype(jnp.int32)[:, None]
        cvs.append(m)
        cis.append(am + base)
        work = jnp.where(lane == am, jnp.inf, work)

    catv = jnp.concatenate([vals[:, 0:3]] + cvs, axis=1)
    cati = jnp.concatenate([inds[:, 0:3]] + cis, axis=1)
    lane6 = lax.broadcasted_iota(jnp.int32, catv.shape, 1)
    nvs, nis = [], []
    for _ in range(3):
        m = jnp.min(catv, axis=1, keepdims=True)
        am = jnp.argmin(catv, axis=1).astype(jnp.int32)[:, None]
        sel = lane6 == am
        nvs.append(m)
        nis.append(jnp.sum(jnp.where(sel, cati, 0), axis=1, keepdims=True))
        catv = jnp.where(sel, jnp.inf, catv)
    vals[:, 0:3] = jnp.concatenate(nvs, axis=1)
    inds[:, 0:3] = jnp.concatenate(nis, axis=1)

    @pl.when(ch == KNN_NCH - 1)
    def _flush():
        sq_ref[:, 0:3] = vals[:, 0:3]
        sq_ref[:, 3:] = jnp.ones((sq_ref.shape[0], 5), jnp.float32)
        idx_ref[:, 0:3] = inds[:, 0:3]
        idx_ref[:, 3:] = jnp.zeros((idx_ref.shape[0], 5), jnp.int32)


def _knn_top3(h_pos, l_posT_ch):
    nqb = N // KNN_QB
    return pl.pallas_call(
        _knn_kernel,
        grid=(nqb, KNN_NCH),
        in_specs=[
            pl.BlockSpec((KNN_QB, 3), lambda qb, ch: (qb, 0)),
            pl.BlockSpec((1, 3, KNN_CB), lambda qb, ch: (ch, 0, 0)),
        ],
        out_specs=[
            pl.BlockSpec((KNN_QB, 8), lambda qb, ch: (qb, 0)),
            pl.BlockSpec((KNN_QB, 8), lambda qb, ch: (qb, 0)),
        ],
        out_shape=[
            jax.ShapeDtypeStruct((N, 8), jnp.float32),
            jax.ShapeDtypeStruct((N, 8), jnp.int32),
        ],
        scratch_shapes=[
            pltpu.VMEM((KNN_QB, 8), jnp.float32),
            pltpu.VMEM((KNN_QB, 8), jnp.int32),
        ],
    )(h_pos, l_posT_ch)


def _interp_kernel(rows_ref, sq_ref, out_ref):
    w = 1.0 / jnp.clip(sq_ref[:, 0:3], 1e-16, None)
    num = w[:, 0:1] * rows_ref[:, 0, :D]
    num = num + w[:, 1:2] * rows_ref[:, 1, :D]
    num = num + w[:, 2:3] * rows_ref[:, 2, :D]
    den = w[:, 0:1] + w[:, 1:2]
    den = den + w[:, 2:3]
    out_ref[:, :D] = num / den
    out_ref[:, D:] = jnp.ones((out_ref.shape[0], DP - D), jnp.float32)


def _interp(rows, sq):
    BM = 1000
    nb = N // BM
    return pl.pallas_call(
        _interp_kernel,
        grid=(nb,),
        in_specs=[
            pl.BlockSpec((BM, 3, DP), lambda i: (i, 0, 0)),
            pl.BlockSpec((BM, 8), lambda i: (i, 0)),
        ],
        out_specs=pl.BlockSpec((BM, DP), lambda i: (i, 0)),
        out_shape=jax.ShapeDtypeStruct((N, DP), jnp.float32),
    )(rows, sq)


def kernel(l_pos1, l_y1, l_e1, h_pos1, h_e1, Wenc1, benc1, Wenc2, benc2,
           Wl1, bl1, Wr1, Wl2, bl2, Wr2):
    sc_scatter = _build_sc_scatter()
    sc_gather = _build_sc_gather()

    zeros = jnp.zeros((ZSTRIPE, DP), jnp.float32)

    def edge_parts(e):
        src = e[0].astype(jnp.int32)
        dst = e[1].astype(jnp.int32)
        pad = NW * EPW - E
        src = jnp.concatenate([src, jnp.zeros((pad,), jnp.int32)])
        dst = jnp.concatenate([dst, jnp.full((pad,), N, jnp.int32)])
        return src.reshape(NW, ECH, 128), dst.reshape(NW, ECH, 128)

    l_src, l_dst = edge_parts(l_e1)
    h_src, h_dst = edge_parts(h_e1)

    xin = jnp.concatenate([l_y1, l_pos1], axis=-1)
    x = _encoder(xin, Wenc1, benc1, Wenc2, benc2)

    for i in range(2):
        p = sc_scatter(x, l_src, l_dst, zeros)
        x = _sage_dense(p, x, Wl1[i], Wr1[i])

    l_posT_ch = l_pos1.T.reshape(3, KNN_NCH, KNN_CB).transpose(1, 0, 2)
    sq, idx = _knn_top3(h_pos1, l_posT_ch)
    idx_flat = idx[:, 0:3].reshape(-1)
    idx_flat = jnp.concatenate(
        [idx_flat, jnp.zeros((GIDX - 3 * N,), jnp.int32)]).reshape(NW, 8, 128)
    rows = sc_gather(x, idx_flat)
    rows = rows[:3 * N].reshape(N, 3, DP)
    x = _interp(rows, sq)

    for i in range(2):
        p = sc_scatter(x, h_src, h_dst, zeros)
        x = _sage_dense(p, x, Wl2[i], Wr2[i])

    return x[:, :D]


# speedup vs baseline: 1.3525x; 1.3525x over previous
import functools

import jax
import jax.numpy as jnp
from jax import lax
from jax.experimental import pallas as pl
from jax.experimental.pallas import tpu as pltpu
from jax.experimental.pallas import tpu_sc as plsc

N = 10000
D = 128
DP = 144
E = 320000
NC, NS = 2, 16
NW = NC * NS
ECH0 = 118
ECH1 = 40
NCHUNK = NS * (ECH0 + ECH1)
AGG_ROWS = 10112
ZSTRIPE = 632

KNN_CB = 2000
KNN_NCH = 5
KNN_QB = 400

GIDX = 32768


def _build_sc_scatter():
    mesh = plsc.VectorSubcoreMesh(core_axis_name="c", subcore_axis_name="s")

    @functools.partial(
        pl.kernel,
        out_type=jax.ShapeDtypeStruct((NC, N, DP), jnp.float32),
        mesh=mesh,
        scratch_types=[
            pltpu.VMEM((2, 128), jnp.int32),
            pltpu.VMEM((2, 128), jnp.int32),
            pltpu.VMEM((2, 128, DP), jnp.float32),
            pltpu.VMEM_SHARED((AGG_ROWS, DP), jnp.float32),
            pltpu.SemaphoreType.DMA,
            pltpu.SemaphoreType.DMA,
            pltpu.SemaphoreType.DMA,
            pltpu.SemaphoreType.DMA,
        ],
        compiler_params=pltpu.CompilerParams(use_tc_tiling_on_sc=False),
    )
    def sc_scatter(table, src_r, dst_r, zeros, out, src_v, dst_v, rows_v,
                   agg_sh, sem_i, sem_g0, sem_g1, sem_s):
        c = lax.axis_index("c")
        s = lax.axis_index("s")
        pltpu.sync_copy(zeros, agg_sh.at[pl.ds(s * ZSTRIPE, ZSTRIPE)])
        plsc.subcore_barrier()

        sem_g = [sem_g0, sem_g1]

        def run(base, nch):
            @pl.loop(0, nch, step=2)
            def _group(j):
                ids = []
                for b in range(2):
                    ids.append(pltpu.async_copy(src_r.at[base + j + b],
                                                src_v.at[b], sem_i))
                    ids.append(pltpu.async_copy(dst_r.at[base + j + b],
                                                dst_v.at[b], sem_i))
                for d in ids:
                    d.wait()
                gds = [pltpu.async_copy(table.at[src_v.at[b]],
                                        rows_v.at[b], sem_g[b])
                       for b in range(2)]
                sds = []
                for b in range(2):
                    gds[b].wait()
                    sds.append(pltpu.async_copy(rows_v.at[b],
                                                agg_sh.at[dst_v.at[b]],
                                                sem_s, add=True))
                for d in sds:
                    d.wait()

        @pl.when(c == 0)
        def _core0():
            run(s * ECH0, ECH0)

        @pl.when(c == 1)
        def _core1():
            run(NS * ECH0 + s * ECH1, ECH1)

        plsc.subcore_barrier()

        @pl.when(s < NS - 1)
        def _copy_full():
            pltpu.sync_copy(agg_sh.at[pl.ds(s * ZSTRIPE, ZSTRIPE)],
                            out.at[c, pl.ds(s * ZSTRIPE, ZSTRIPE)])

        @pl.when(s == NS - 1)
        def _copy_tail():
            pltpu.sync_copy(agg_sh.at[pl.ds((NS - 1) * ZSTRIPE, N - (NS - 1) * ZSTRIPE)],
                            out.at[c, pl.ds((NS - 1) * ZSTRIPE, N - (NS - 1) * ZSTRIPE)])

    return sc_scatter


def _build_sc_gather():
    mesh = plsc.VectorSubcoreMesh(core_axis_name="c", subcore_axis_name="s")

    @functools.partial(
        pl.kernel,
        out_type=jax.ShapeDtypeStruct((GIDX, DP), jnp.float32),
        mesh=mesh,
        scratch_types=[
            pltpu.VMEM((12, 128), jnp.int32),
            pltpu.VMEM((2, 128, DP), jnp.float32),
            pltpu.SemaphoreType.DMA,
            pltpu.SemaphoreType.DMA,
            pltpu.SemaphoreType.DMA,
        ],
        compiler_params=pltpu.CompilerParams(use_tc_tiling_on_sc=False),
    )
    def sc_gather(table, idx_r, out, idx_v, rows_v, sem_g0, sem_g1, sem_s):
        c = lax.axis_index("c")
        s = lax.axis_index("s")
        sem_g = [sem_g0, sem_g1]

        cbase = jnp.where(c == 0, s * 12, NS * 12 + s * 4)

        def run(nch):
            pltpu.sync_copy(idx_r.at[pl.ds(cbase, nch)],
                            idx_v.at[pl.ds(0, nch)])

            @pl.loop(0, nch, step=2)
            def _group(j):
                gds = [pltpu.async_copy(table.at[idx_v.at[j + b]],
                                        rows_v.at[b], sem_g[b])
                       for b in range(2)]
                sds = []
                for b in range(2):
                    gds[b].wait()
                    sds.append(pltpu.async_copy(
                        rows_v.at[b],
                        out.at[pl.ds((cbase + j + b) * 128, 128)], sem_s))
                for d in sds:
                    d.wait()

        @pl.when(c == 0)
        def _core0():
            run(12)

        @pl.when(c == 1)
        def _core1():
            run(4)

    return sc_gather


def _enc_kernel(x_ref, w1_ref, b1_ref, w2_ref, b2_ref, out_ref):
    x = x_ref[...]
    h = lax.dot_general(x, w1_ref[...], (((1,), (1,)), ((), ())),
                        preferred_element_type=jnp.float32) + b1_ref[...]
    h = jnp.maximum(h, 0.0)
    y = lax.dot_general(h, w2_ref[...], (((1,), (1,)), ((), ())),
                        preferred_element_type=jnp.float32) + b2_ref[...]
    out_ref[:, :D] = y
    out_ref[:, D:] = jnp.ones((out_ref.shape[0], DP - D), jnp.float32)


def _encoder(xin, W1, b1, W2, b2):
    BM = 1000
    nb = N // BM
    return pl.pallas_call(
        _enc_kernel,
        grid=(nb,),
        in_specs=[
            pl.BlockSpec((BM, xin.shape[1]), lambda i: (i, 0)),
            pl.BlockSpec(W1.shape, lambda i: (0, 0)),
            pl.BlockSpec((1, D), lambda i: (0, 0)),
            pl.BlockSpec(W2.shape, lambda i: (0, 0)),
            pl.BlockSpec((1, D), lambda i: (0, 0)),
        ],
        out_specs=pl.BlockSpec((BM, DP), lambda i: (i, 0)),
        out_shape=jax.ShapeDtypeStruct((N, DP), jnp.float32),
    )(xin, W1, b1.reshape(1, D), W2, b2.reshape(1, D))


def _sage_dense_kernel(p_ref, x_ref, wl_ref, wr_ref, out_ref,
                       acc_agg, acc_y, yraw, *, bm):
    ph = pl.program_id(0)
    i = pl.program_id(1)

    @pl.when(jnp.logical_and(ph == 0, i == 0))
    def _init():
        acc_agg[...] = jnp.zeros_like(acc_agg)
        acc_y[...] = jnp.zeros_like(acc_y)

    def _agg_block():
        feat = p_ref[0, :, :D] + p_ref[1, :, :D]
        cnt = p_ref[0, :, D:D + 1] + p_ref[1, :, D:D + 1]
        return feat / jnp.clip(cnt, 1.0, None)

    @pl.when(ph == 0)
    def _phase0():
        acc_agg[...] += jnp.sum(_agg_block(), axis=0, keepdims=True)

    @pl.when(ph == 1)
    def _phase1():
        agg = _agg_block() - acc_agg[...] / float(N)
        xf = x_ref[:, :D]
        t = lax.dot_general(agg, wl_ref[...], (((1,), (1,)), ((), ())),
                            preferred_element_type=jnp.float32)
        t = t + lax.dot_general(xf, wr_ref[...], (((1,), (1,)), ((), ())),
                                preferred_element_type=jnp.float32)
        yr = xf + jnp.maximum(t, 0.0)
        acc_y[...] += jnp.sum(yr, axis=0, keepdims=True)
        yraw[pl.ds(i * bm, bm), :] = yr

    @pl.when(ph == 2)
    def _phase2():
        out_ref[:, :D] = yraw[pl.ds(i * bm, bm), :] - acc_y[...] / float(N)
        out_ref[:, D:] = jnp.ones((bm, DP - D), jnp.float32)


def _sage_dense(p, x, Wl, Wr):
    BM = 1000
    nb = N // BM
    return pl.pallas_call(
        functools.partial(_sage_dense_kernel, bm=BM),
        grid=(3, nb),
        in_specs=[
            pl.BlockSpec((2, BM, DP), lambda ph, i: (0, i, 0)),
            pl.BlockSpec((BM, DP), lambda ph, i: (i, 0)),
            pl.BlockSpec((D, D), lambda ph, i: (0, 0)),
            pl.BlockSpec((D, D), lambda ph, i: (0, 0)),
        ],
        out_specs=pl.BlockSpec((BM, DP), lambda ph, i: (i, 0)),
        out_shape=jax.ShapeDtypeStruct((N, DP), jnp.float32),
        scratch_shapes=[
            pltpu.VMEM((1, D), jnp.float32),
            pltpu.VMEM((1, D), jnp.float32),
            pltpu.VMEM((N, D), jnp.float32),
        ],
    )(p, x, Wl, Wr)


def _knn_kernel(q_ref, c_ref, sq_ref, idx_ref, vals, inds):
    ch = pl.program_id(1)

    @pl.when(ch == 0)
    def _init():
        vals[...] = jnp.full_like(vals[...], jnp.inf)
        inds[...] = jnp.zeros_like(inds[...])

    q = q_ref[...]
    c = c_ref[0]
    d2 = (q[:, 0:1] - c[0:1, :]) ** 2
    d2 = d2 + (q[:, 1:2] - c[1:2, :]) ** 2
    d2 = d2 + (q[:, 2:3] - c[2:3, :]) ** 2

    base = ch * KNN_CB
    lane = lax.broadcasted_iota(jnp.int32, d2.shape, 1)
    cvs, cis = [], []
    work = d2
    for _ in range(3):
        m = jnp.min(work, axis=1, keepdims=True)
        am = jnp.argmin(work, axis=1).astype(jnp.int32)[:, None]
        cvs.append(m)
        cis.append(am + base)
        work = jnp.where(lane == am, jnp.inf, work)

    catv = jnp.concatenate([vals[:, 0:3]] + cvs, axis=1)
    cati = jnp.concatenate([inds[:, 0:3]] + cis, axis=1)
    lane6 = lax.broadcasted_iota(jnp.int32, catv.shape, 1)
    nvs, nis = [], []
    for _ in range(3):
        m = jnp.min(catv, axis=1, keepdims=True)
        am = jnp.argmin(catv, axis=1).astype(jnp.int32)[:, None]
        sel = lane6 == am
        nvs.append(m)
        nis.append(jnp.sum(jnp.where(sel, cati, 0), axis=1, keepdims=True))
        catv = jnp.where(sel, jnp.inf, catv)
    vals[:, 0:3] = jnp.concatenate(nvs, axis=1)
    inds[:, 0:3] = jnp.concatenate(nis, axis=1)

    @pl.when(ch == KNN_NCH - 1)
    def _flush():
        sq_ref[:, 0:3] = vals[:, 0:3]
        sq_ref[:, 3:] = jnp.ones((sq_ref.shape[0], 5), jnp.float32)
        idx_ref[:, 0:3] = inds[:, 0:3]
        idx_ref[:, 3:] = jnp.zeros((idx_ref.shape[0], 5), jnp.int32)


def _knn_top3(h_pos, l_posT_ch):
    nqb = N // KNN_QB
    return pl.pallas_call(
        _knn_kernel,
        grid=(nqb, KNN_NCH),
        in_specs=[
            pl.BlockSpec((KNN_QB, 3), lambda qb, ch: (qb, 0)),
            pl.BlockSpec((1, 3, KNN_CB), lambda qb, ch: (ch, 0, 0)),
        ],
        out_specs=[
            pl.BlockSpec((KNN_QB, 8), lambda qb, ch: (qb, 0)),
            pl.BlockSpec((KNN_QB, 8), lambda qb, ch: (qb, 0)),
        ],
        out_shape=[
            jax.ShapeDtypeStruct((N, 8), jnp.float32),
            jax.ShapeDtypeStruct((N, 8), jnp.int32),
        ],
        scratch_shapes=[
            pltpu.VMEM((KNN_QB, 8), jnp.float32),
            pltpu.VMEM((KNN_QB, 8), jnp.int32),
        ],
    )(h_pos, l_posT_ch)


def _interp_kernel(rows_ref, sq_ref, out_ref):
    w = 1.0 / jnp.clip(sq_ref[:, 0:3], 1e-16, None)
    num = w[:, 0:1] * rows_ref[:, 0, :D]
    num = num + w[:, 1:2] * rows_ref[:, 1, :D]
    num = num + w[:, 2:3] * rows_ref[:, 2, :D]
    den = w[:, 0:1] + w[:, 1:2]
    den = den + w[:, 2:3]
    out_ref[:, :D] = num / den
    out_ref[:, D:] = jnp.ones((out_ref.shape[0], DP - D), jnp.float32)


def _interp(rows, sq):
    BM = 1000
    nb = N // BM
    return pl.pallas_call(
        _interp_kernel,
        grid=(nb,),
        in_specs=[
            pl.BlockSpec((BM, 3, DP), lambda i: (i, 0, 0)),
            pl.BlockSpec((BM, 8), lambda i: (i, 0)),
        ],
        out_specs=pl.BlockSpec((BM, DP), lambda i: (i, 0)),
        out_shape=jax.ShapeDtypeStruct((N, DP), jnp.float32),
    )(rows, sq)


def kernel(l_pos1, l_y1, l_e1, h_pos1, h_e1, Wenc1, benc1, Wenc2, benc2,
           Wl1, bl1, Wr1, Wl2, bl2, Wr2):
    sc_scatter = _build_sc_scatter()
    sc_gather = _build_sc_gather()

    zeros = jnp.zeros((ZSTRIPE, DP), jnp.float32)

    def edge_parts(e):
        src = e[0].astype(jnp.int32)
        dst = e[1].astype(jnp.int32)
        pad = NCHUNK * 128 - E
        src = jnp.concatenate([src, jnp.zeros((pad,), jnp.int32)])
        dst = jnp.concatenate([dst, jnp.full((pad,), N, jnp.int32)])
        return src.reshape(NCHUNK, 128), dst.reshape(NCHUNK, 128)

    l_src, l_dst = edge_parts(l_e1)
    h_src, h_dst = edge_parts(h_e1)

    xin = jnp.concatenate([l_y1, l_pos1], axis=-1)
    x = _encoder(xin, Wenc1, benc1, Wenc2, benc2)

    for i in range(2):
        p = sc_scatter(x, l_src, l_dst, zeros)
        x = _sage_dense(p, x, Wl1[i], Wr1[i])

    l_posT_ch = l_pos1.T.reshape(3, KNN_NCH, KNN_CB).transpose(1, 0, 2)
    sq, idx = _knn_top3(h_pos1, l_posT_ch)
    idx_flat = idx[:, 0:3].reshape(-1)
    idx_flat = jnp.concatenate(
        [idx_flat, jnp.zeros((GIDX - 3 * N,), jnp.int32)]).reshape(256, 128)
    rows = sc_gather(x, idx_flat)
    rows = rows[:3 * N].reshape(N, 3, DP)
    x = _interp(rows, sq)

    for i in range(2):
        p = sc_scatter(x, h_src, h_dst, zeros)
        x = _sage_dense(p, x, Wl2[i], Wr2[i])

    return x[:, :D]


# speedup vs baseline: 1.4384x; 1.0636x over previous
import functools

import jax
import jax.numpy as jnp
from jax import lax
from jax.experimental import pallas as pl
from jax.experimental.pallas import tpu as pltpu
from jax.experimental.pallas import tpu_sc as plsc

N = 10000
D = 128
DP = 144
E = 320000
NC, NS = 2, 16
NW = NC * NS
ECH0 = 118
ECH1 = 40
NCHUNK = NS * (ECH0 + ECH1)
AGG_ROWS = 10112
ZSTRIPE = 632

KNN_CB = 2000
KNN_NCH = 5
KNN_QB = 400

GCH0 = 14
GCH1 = 1
GNCH = NS * (GCH0 + GCH1)
GIDX = GNCH * 128


def _build_sc_scatter():
    mesh = plsc.VectorSubcoreMesh(core_axis_name="c", subcore_axis_name="s")

    @functools.partial(
        pl.kernel,
        out_type=jax.ShapeDtypeStruct((NC, N, DP), jnp.float32),
        mesh=mesh,
        scratch_types=[
            pltpu.VMEM((2, 128), jnp.int32),
            pltpu.VMEM((2, 128), jnp.int32),
            pltpu.VMEM((2, 128, DP), jnp.float32),
            pltpu.VMEM_SHARED((AGG_ROWS, DP), jnp.float32),
            pltpu.SemaphoreType.DMA,
            pltpu.SemaphoreType.DMA,
            pltpu.SemaphoreType.DMA,
            pltpu.SemaphoreType.DMA,
        ],
        compiler_params=pltpu.CompilerParams(use_tc_tiling_on_sc=False),
    )
    def sc_scatter(table, src_r, dst_r, zeros, out, src_v, dst_v, rows_v,
                   agg_sh, sem_i, sem_g0, sem_g1, sem_s):
        c = lax.axis_index("c")
        s = lax.axis_index("s")
        pltpu.sync_copy(zeros, agg_sh.at[pl.ds(s * ZSTRIPE, ZSTRIPE)])
        plsc.subcore_barrier()

        sem_g = [sem_g0, sem_g1]

        def run(base, nch):
            @pl.loop(0, nch, step=2)
            def _group(j):
                ids = []
                for b in range(2):
                    ids.append(pltpu.async_copy(src_r.at[base + j + b],
                                                src_v.at[b], sem_i))
                    ids.append(pltpu.async_copy(dst_r.at[base + j + b],
                                                dst_v.at[b], sem_i))
                for d in ids:
                    d.wait()
                gds = [pltpu.async_copy(table.at[src_v.at[b]],
                                        rows_v.at[b], sem_g[b])
                       for b in range(2)]
                sds = []
                for b in range(2):
                    gds[b].wait()
                    sds.append(pltpu.async_copy(rows_v.at[b],
                                                agg_sh.at[dst_v.at[b]],
                                                sem_s, add=True))
                for d in sds:
                    d.wait()

        @pl.when(c == 0)
        def _core0():
            run(s * ECH0, ECH0)

        @pl.when(c == 1)
        def _core1():
            run(NS * ECH0 + s * ECH1, ECH1)

        plsc.subcore_barrier()

        @pl.when(s < NS - 1)
        def _copy_full():
            pltpu.sync_copy(agg_sh.at[pl.ds(s * ZSTRIPE, ZSTRIPE)],
                            out.at[c, pl.ds(s * ZSTRIPE, ZSTRIPE)])

        @pl.when(s == NS - 1)
        def _copy_tail():
            pltpu.sync_copy(agg_sh.at[pl.ds((NS - 1) * ZSTRIPE, N - (NS - 1) * ZSTRIPE)],
                            out.at[c, pl.ds((NS - 1) * ZSTRIPE, N - (NS - 1) * ZSTRIPE)])

    return sc_scatter


def _build_sc_gather():
    mesh = plsc.VectorSubcoreMesh(core_axis_name="c", subcore_axis_name="s")

    @functools.partial(
        pl.kernel,
        out_type=jax.ShapeDtypeStruct((GIDX, DP), jnp.float32),
        mesh=mesh,
        scratch_types=[
            pltpu.VMEM((GCH0, 128), jnp.int32),
            pltpu.VMEM((2, 128, DP), jnp.float32),
            pltpu.SemaphoreType.DMA,
            pltpu.SemaphoreType.DMA,
            pltpu.SemaphoreType.DMA,
        ],
        compiler_params=pltpu.CompilerParams(use_tc_tiling_on_sc=False),
    )
    def sc_gather(table, idx_r, out, idx_v, rows_v, sem_g0, sem_g1, sem_s):
        c = lax.axis_index("c")
        s = lax.axis_index("s")
        sem_g = [sem_g0, sem_g1]

        cbase = jnp.where(c == 0, s * GCH0, NS * GCH0 + s * GCH1)

        @pl.when(c == 0)
        def _core0():
            pltpu.sync_copy(idx_r.at[pl.ds(cbase, GCH0)], idx_v)

            @pl.loop(0, GCH0, step=2)
            def _group(j):
                gds = [pltpu.async_copy(table.at[idx_v.at[j + b]],
                                        rows_v.at[b], sem_g[b])
                       for b in range(2)]
                sds = []
                for b in range(2):
                    gds[b].wait()
                    sds.append(pltpu.async_copy(
                        rows_v.at[b],
                        out.at[pl.ds((cbase + j + b) * 128, 128)], sem_s))
                for d in sds:
                    d.wait()

        @pl.when(c == 1)
        def _core1():
            pltpu.sync_copy(idx_r.at[pl.ds(cbase, GCH1)],
                            idx_v.at[pl.ds(0, GCH1)])
            pltpu.async_copy(table.at[idx_v.at[0]], rows_v.at[0],
                             sem_g0).wait()
            pltpu.sync_copy(rows_v.at[0], out.at[pl.ds(cbase * 128, 128)])

    return sc_gather


def _enc_kernel(x_ref, w1_ref, b1_ref, w2_ref, b2_ref, out_ref):
    x = x_ref[...]
    h = lax.dot_general(x, w1_ref[...], (((1,), (1,)), ((), ())),
                        preferred_element_type=jnp.float32) + b1_ref[...]
    h = jnp.maximum(h, 0.0)
    y = lax.dot_general(h, w2_ref[...], (((1,), (1,)), ((), ())),
                        preferred_element_type=jnp.float32) + b2_ref[...]
    out_ref[:, :D] = y
    out_ref[:, D:] = jnp.ones((out_ref.shape[0], DP - D), jnp.float32)


def _encoder(xin, W1, b1, W2, b2):
    BM = 1000
    nb = N // BM
    return pl.pallas_call(
        _enc_kernel,
        grid=(nb,),
        in_specs=[
            pl.BlockSpec((BM, xin.shape[1]), lambda i: (i, 0)),
            pl.BlockSpec(W1.shape, lambda i: (0, 0)),
            pl.BlockSpec((1, D), lambda i: (0, 0)),
            pl.BlockSpec(W2.shape, lambda i: (0, 0)),
            pl.BlockSpec((1, D), lambda i: (0, 0)),
        ],
        out_specs=pl.BlockSpec((BM, DP), lambda i: (i, 0)),
        out_shape=jax.ShapeDtypeStruct((N, DP), jnp.float32),
    )(xin, W1, b1.reshape(1, D), W2, b2.reshape(1, D))


def _sage_dense_kernel(p_ref, x_ref, wl_ref, wr_ref, out_ref,
                       acc_agg, acc_y, yraw, *, bm):
    ph = pl.program_id(0)
    i = pl.program_id(1)

    @pl.when(jnp.logical_and(ph == 0, i == 0))
    def _init():
        acc_agg[...] = jnp.zeros_like(acc_agg)
        acc_y[...] = jnp.zeros_like(acc_y)

    def _agg_block():
        feat = p_ref[0, :, :D] + p_ref[1, :, :D]
        cnt = p_ref[0, :, D:D + 1] + p_ref[1, :, D:D + 1]
        return feat / jnp.clip(cnt, 1.0, None)

    @pl.when(ph == 0)
    def _phase0():
        acc_agg[...] += jnp.sum(_agg_block(), axis=0, keepdims=True)

    @pl.when(ph == 1)
    def _phase1():
        agg = _agg_block() - acc_agg[...] / float(N)
        xf = x_ref[:, :D]
        t = lax.dot_general(agg, wl_ref[...], (((1,), (1,)), ((), ())),
                            preferred_element_type=jnp.float32)
        t = t + lax.dot_general(xf, wr_ref[...], (((1,), (1,)), ((), ())),
                                preferred_element_type=jnp.float32)
        yr = xf + jnp.maximum(t, 0.0)
        acc_y[...] += jnp.sum(yr, axis=0, keepdims=True)
        yraw[pl.ds(i * bm, bm), :] = yr

    @pl.when(ph == 2)
    def _phase2():
        out_ref[:, :D] = yraw[pl.ds(i * bm, bm), :] - acc_y[...] / float(N)
        out_ref[:, D:] = jnp.ones((bm, DP - D), jnp.float32)


def _sage_dense(p, x, Wl, Wr):
    BM = 1000
    nb = N // BM
    return pl.pallas_call(
        functools.partial(_sage_dense_kernel, bm=BM),
        grid=(3, nb),
        in_specs=[
            pl.BlockSpec((2, BM, DP), lambda ph, i: (0, i, 0)),
            pl.BlockSpec((BM, DP), lambda ph, i: (i, 0)),
            pl.BlockSpec((D, D), lambda ph, i: (0, 0)),
            pl.BlockSpec((D, D), lambda ph, i: (0, 0)),
        ],
        out_specs=pl.BlockSpec((BM, DP), lambda ph, i: (i, 0)),
        out_shape=jax.ShapeDtypeStruct((N, DP), jnp.float32),
        scratch_shapes=[
            pltpu.VMEM((1, D), jnp.float32),
            pltpu.VMEM((1, D), jnp.float32),
            pltpu.VMEM((N, D), jnp.float32),
        ],
    )(p, x, Wl, Wr)


def _knn_kernel(q_ref, c_ref, sq_ref, idx_ref, vals, inds):
    ch = pl.program_id(1)

    @pl.when(ch == 0)
    def _init():
        vals[...] = jnp.full_like(vals[...], jnp.inf)
        inds[...] = jnp.zeros_like(inds[...])

    q = q_ref[...]
    c = c_ref[0]
    d2 = (q[:, 0:1] - c[0:1, :]) ** 2
    d2 = d2 + (q[:, 1:2] - c[1:2, :]) ** 2
    d2 = d2 + (q[:, 2:3] - c[2:3, :]) ** 2

    base = ch * KNN_CB
    lane = lax.broadcasted_iota(jnp.int32, d2.shape, 1)
    cvs, cis = [], []
    work = d2
    for _ in range(3):
        m = jnp.min(work, axis=1, keepdims=True)
        am = jnp.argmin(work, axis=1).astype(jnp.int32)[:, None]
        cvs.append(m)
        cis.append(am + base)
        work = jnp.where(lane == am, jnp.inf, work)

    catv = jnp.concatenate([vals[:, 0:3]] + cvs, axis=1)
    cati = jnp.concatenate([inds[:, 0:3]] + cis, axis=1)
    lane6 = lax.broadcasted_iota(jnp.int32, catv.shape, 1)
    nvs, nis = [], []
    for _ in range(3):
        m = jnp.min(catv, axis=1, keepdims=True)
        am = jnp.argmin(catv, axis=1).astype(jnp.int32)[:, None]
        sel = lane6 == am
        nvs.append(m)
        nis.append(jnp.sum(jnp.where(sel, cati, 0), axis=1, keepdims=True))
        catv = jnp.where(sel, jnp.inf, catv)
    vals[:, 0:3] = jnp.concatenate(nvs, axis=1)
    inds[:, 0:3] = jnp.concatenate(nis, axis=1)

    @pl.when(ch == KNN_NCH - 1)
    def _flush():
        sq_ref[:, 0:3] = vals[:, 0:3]
        sq_ref[:, 3:] = jnp.ones((sq_ref.shape[0], 5), jnp.float32)
        idx_ref[:, 0:3] = inds[:, 0:3]
        idx_ref[:, 3:] = jnp.zeros((idx_ref.shape[0], 5), jnp.int32)


def _knn_top3(h_pos, l_posT_ch):
    nqb = N // KNN_QB
    return pl.pallas_call(
        _knn_kernel,
        grid=(nqb, KNN_NCH),
        in_specs=[
            pl.BlockSpec((KNN_QB, 3), lambda qb, ch: (qb, 0)),
            pl.BlockSpec((1, 3, KNN_CB), lambda qb, ch: (ch, 0, 0)),
        ],
        out_specs=[
            pl.BlockSpec((KNN_QB, 8), lambda qb, ch: (qb, 0)),
            pl.BlockSpec((KNN_QB, 8), lambda qb, ch: (qb, 0)),
        ],
        out_shape=[
            jax.ShapeDtypeStruct((N, 8), jnp.float32),
            jax.ShapeDtypeStruct((N, 8), jnp.int32),
        ],
        scratch_shapes=[
            pltpu.VMEM((KNN_QB, 8), jnp.float32),
            pltpu.VMEM((KNN_QB, 8), jnp.int32),
        ],
    )(h_pos, l_posT_ch)


def _interp_kernel(rows_ref, sq_ref, out_ref):
    w = 1.0 / jnp.clip(sq_ref[:, 0:3], 1e-16, None)
    num = w[:, 0:1] * rows_ref[:, 0, :D]
    num = num + w[:, 1:2] * rows_ref[:, 1, :D]
    num = num + w[:, 2:3] * rows_ref[:, 2, :D]
    den = w[:, 0:1] + w[:, 1:2]
    den = den + w[:, 2:3]
    out_ref[:, :D] = num / den
    out_ref[:, D:] = jnp.ones((out_ref.shape[0], DP - D), jnp.float32)


def _interp(rows, sq):
    BM = 1000
    nb = N // BM
    return pl.pallas_call(
        _interp_kernel,
        grid=(nb,),
        in_specs=[
            pl.BlockSpec((BM, 3, DP), lambda i: (i, 0, 0)),
            pl.BlockSpec((BM, 8), lambda i: (i, 0)),
        ],
        out_specs=pl.BlockSpec((BM, DP), lambda i: (i, 0)),
        out_shape=jax.ShapeDtypeStruct((N, DP), jnp.float32),
    )(rows, sq)


def kernel(l_pos1, l_y1, l_e1, h_pos1, h_e1, Wenc1, benc1, Wenc2, benc2,
           Wl1, bl1, Wr1, Wl2, bl2, Wr2):
    sc_scatter = _build_sc_scatter()
    sc_gather = _build_sc_gather()

    zeros = jnp.zeros((ZSTRIPE, DP), jnp.float32)

    def edge_parts(e):
        src = e[0].astype(jnp.int32)
        dst = e[1].astype(jnp.int32)
        pad = NCHUNK * 128 - E
        src = jnp.concatenate([src, jnp.zeros((pad,), jnp.int32)])
        dst = jnp.concatenate([dst, jnp.full((pad,), N, jnp.int32)])
        return src.reshape(NCHUNK, 128), dst.reshape(NCHUNK, 128)

    l_src, l_dst = edge_parts(l_e1)
    h_src, h_dst = edge_parts(h_e1)

    xin = jnp.concatenate([l_y1, l_pos1], axis=-1)
    x = _encoder(xin, Wenc1, benc1, Wenc2, benc2)

    for i in range(2):
        p = sc_scatter(x, l_src, l_dst, zeros)
        x = _sage_dense(p, x, Wl1[i], Wr1[i])

    l_posT_ch = l_pos1.T.reshape(3, KNN_NCH, KNN_CB).transpose(1, 0, 2)
    sq, idx = _knn_top3(h_pos1, l_posT_ch)
    idx_flat = idx[:, 0:3].reshape(-1)
    idx_flat = jnp.concatenate(
        [idx_flat, jnp.zeros((GIDX - 3 * N,), jnp.int32)]).reshape(GNCH, 128)
    rows = sc_gather(x, idx_flat).reshape(GIDX // 3, 3, DP)
    x = _interp(rows, sq)

    for i in range(2):
        p = sc_scatter(x, h_src, h_dst, zeros)
        x = _sage_dense(p, x, Wl2[i], Wr2[i])

    return x[:, :D]


# speedup vs baseline: 1.6134x; 1.1216x over previous
import functools

import jax
import jax.numpy as jnp
from jax import lax
from jax.experimental import pallas as pl
from jax.experimental.pallas import tpu as pltpu
from jax.experimental.pallas import tpu_sc as plsc

N = 10000
D = 128
DP = 144
E = 320000
NC, NS = 2, 16
NW = NC * NS
ECH0 = 118
ECH1 = 40
NCHUNK = NS * (ECH0 + ECH1)
AGG_ROWS = 10112
ZSTRIPE = 632

KNN_CB = 2000
KNN_NCH = 5
KNN_QB = 400

GCH0 = 14
GCH1 = 1
GNCH = NS * (GCH0 + GCH1)
GIDX = GNCH * 128


def _build_sc_scatter():
    mesh = plsc.VectorSubcoreMesh(core_axis_name="c", subcore_axis_name="s")

    @functools.partial(
        pl.kernel,
        out_type=jax.ShapeDtypeStruct((NC, N, DP), jnp.float32),
        mesh=mesh,
        scratch_types=[
            pltpu.VMEM((2, 128), jnp.int32),
            pltpu.VMEM((2, 128), jnp.int32),
            pltpu.VMEM((2, 128, DP), jnp.float32),
            pltpu.VMEM_SHARED((AGG_ROWS, DP), jnp.float32),
            pltpu.SemaphoreType.DMA,
            pltpu.SemaphoreType.DMA,
            pltpu.SemaphoreType.DMA,
            pltpu.SemaphoreType.DMA,
        ],
        compiler_params=pltpu.CompilerParams(use_tc_tiling_on_sc=False),
    )
    def sc_scatter(table, src_r, dst_r, zeros, out, src_v, dst_v, rows_v,
                   agg_sh, sem_i, sem_g0, sem_g1, sem_s):
        c = lax.axis_index("c")
        s = lax.axis_index("s")
        pltpu.sync_copy(zeros, agg_sh.at[pl.ds(s * ZSTRIPE, ZSTRIPE)])
        plsc.subcore_barrier()

        sem_g = [sem_g0, sem_g1]

        def run(base, nch):
            @pl.loop(0, nch, step=2)
            def _group(j):
                ids = []
                for b in range(2):
                    ids.append(pltpu.async_copy(src_r.at[base + j + b],
                                                src_v.at[b], sem_i))
                    ids.append(pltpu.async_copy(dst_r.at[base + j + b],
                                                dst_v.at[b], sem_i))
                for d in ids:
                    d.wait()
                gds = [pltpu.async_copy(table.at[src_v.at[b]],
                                        rows_v.at[b], sem_g[b])
                       for b in range(2)]
                sds = []
                for b in range(2):
                    gds[b].wait()
                    sds.append(pltpu.async_copy(rows_v.at[b],
                                                agg_sh.at[dst_v.at[b]],
                                                sem_s, add=True))
                for d in sds:
                    d.wait()

        @pl.when(c == 0)
        def _core0():
            run(s * ECH0, ECH0)

        @pl.when(c == 1)
        def _core1():
            run(NS * ECH0 + s * ECH1, ECH1)

        plsc.subcore_barrier()

        @pl.when(s < NS - 1)
        def _copy_full():
            pltpu.sync_copy(agg_sh.at[pl.ds(s * ZSTRIPE, ZSTRIPE)],
                            out.at[c, pl.ds(s * ZSTRIPE, ZSTRIPE)])

        @pl.when(s == NS - 1)
        def _copy_tail():
            pltpu.sync_copy(agg_sh.at[pl.ds((NS - 1) * ZSTRIPE, N - (NS - 1) * ZSTRIPE)],
                            out.at[c, pl.ds((NS - 1) * ZSTRIPE, N - (NS - 1) * ZSTRIPE)])

    return sc_scatter


def _build_sc_gather():
    mesh = plsc.VectorSubcoreMesh(core_axis_name="c", subcore_axis_name="s")

    @functools.partial(
        pl.kernel,
        out_type=jax.ShapeDtypeStruct((GIDX, DP), jnp.float32),
        mesh=mesh,
        scratch_types=[
            pltpu.VMEM((GCH0, 128), jnp.int32),
            pltpu.VMEM((2, 128, DP), jnp.float32),
            pltpu.SemaphoreType.DMA,
            pltpu.SemaphoreType.DMA,
            pltpu.SemaphoreType.DMA,
        ],
        compiler_params=pltpu.CompilerParams(use_tc_tiling_on_sc=False),
    )
    def sc_gather(table, idx_r, out, idx_v, rows_v, sem_g0, sem_g1, sem_s):
        c = lax.axis_index("c")
        s = lax.axis_index("s")
        sem_g = [sem_g0, sem_g1]

        cbase = jnp.where(c == 0, s * GCH0, NS * GCH0 + s * GCH1)

        @pl.when(c == 0)
        def _core0():
            pltpu.sync_copy(idx_r.at[pl.ds(cbase, GCH0)], idx_v)

            @pl.loop(0, GCH0, step=2)
            def _group(j):
                gds = [pltpu.async_copy(table.at[idx_v.at[j + b]],
                                        rows_v.at[b], sem_g[b])
                       for b in range(2)]
                sds = []
                for b in range(2):
                    gds[b].wait()
                    sds.append(pltpu.async_copy(
                        rows_v.at[b],
                        out.at[pl.ds((cbase + j + b) * 128, 128)], sem_s))
                for d in sds:
                    d.wait()

        @pl.when(c == 1)
        def _core1():
            pltpu.sync_copy(idx_r.at[pl.ds(cbase, GCH1)],
                            idx_v.at[pl.ds(0, GCH1)])
            pltpu.async_copy(table.at[idx_v.at[0]], rows_v.at[0],
                             sem_g0).wait()
            pltpu.sync_copy(rows_v.at[0], out.at[pl.ds(cbase * 128, 128)])

    return sc_gather


def _enc_kernel(x_ref, w1_ref, b1_ref, w2_ref, b2_ref, out_ref):
    x = x_ref[...]
    h = lax.dot_general(x, w1_ref[...], (((1,), (1,)), ((), ())),
                        preferred_element_type=jnp.float32) + b1_ref[...]
    h = jnp.maximum(h, 0.0)
    y = lax.dot_general(h, w2_ref[...], (((1,), (1,)), ((), ())),
                        preferred_element_type=jnp.float32) + b2_ref[...]
    out_ref[:, :D] = y
    out_ref[:, D:] = jnp.ones((out_ref.shape[0], DP - D), jnp.float32)


def _encoder(xin, W1, b1, W2, b2):
    BM = 1000
    nb = N // BM
    return pl.pallas_call(
        _enc_kernel,
        grid=(nb,),
        in_specs=[
            pl.BlockSpec((BM, xin.shape[1]), lambda i: (i, 0)),
            pl.BlockSpec(W1.shape, lambda i: (0, 0)),
            pl.BlockSpec((1, D), lambda i: (0, 0)),
            pl.BlockSpec(W2.shape, lambda i: (0, 0)),
            pl.BlockSpec((1, D), lambda i: (0, 0)),
        ],
        out_specs=pl.BlockSpec((BM, DP), lambda i: (i, 0)),
        out_shape=jax.ShapeDtypeStruct((N, DP), jnp.float32),
    )(xin, W1, b1.reshape(1, D), W2, b2.reshape(1, D))


def _sage_dense_kernel(p_ref, x_ref, wl_ref, wr_ref, out_ref,
                       acc_agg, acc_y, yraw, *, bm):
    ph = pl.program_id(0)
    i = pl.program_id(1)

    @pl.when(jnp.logical_and(ph == 0, i == 0))
    def _init():
        acc_agg[...] = jnp.zeros_like(acc_agg)
        acc_y[...] = jnp.zeros_like(acc_y)

    def _agg_block():
        feat = p_ref[0, :, :D] + p_ref[1, :, :D]
        cnt = p_ref[0, :, D:D + 1] + p_ref[1, :, D:D + 1]
        return feat / jnp.clip(cnt, 1.0, None)

    @pl.when(ph == 0)
    def _phase0():
        acc_agg[...] += jnp.sum(_agg_block(), axis=0, keepdims=True)

    @pl.when(ph == 1)
    def _phase1():
        agg = _agg_block() - acc_agg[...] / float(N)
        xf = x_ref[:, :D]
        t = lax.dot_general(agg, wl_ref[...], (((1,), (1,)), ((), ())),
                            preferred_element_type=jnp.float32)
        t = t + lax.dot_general(xf, wr_ref[...], (((1,), (1,)), ((), ())),
                                preferred_element_type=jnp.float32)
        yr = xf + jnp.maximum(t, 0.0)
        acc_y[...] += jnp.sum(yr, axis=0, keepdims=True)
        yraw[pl.ds(i * bm, bm), :] = yr

    @pl.when(ph == 2)
    def _phase2():
        out_ref[:, :D] = yraw[pl.ds(i * bm, bm), :] - acc_y[...] / float(N)
        out_ref[:, D:] = jnp.ones((bm, DP - D), jnp.float32)


def _sage_dense(p, x, Wl, Wr):
    BM = 1000
    nb = N // BM
    return pl.pallas_call(
        functools.partial(_sage_dense_kernel, bm=BM),
        grid=(3, nb),
        in_specs=[
            pl.BlockSpec((2, BM, DP), lambda ph, i: (0, i, 0)),
            pl.BlockSpec((BM, DP), lambda ph, i: (i, 0)),
            pl.BlockSpec((D, D), lambda ph, i: (0, 0)),
            pl.BlockSpec((D, D), lambda ph, i: (0, 0)),
        ],
        out_specs=pl.BlockSpec((BM, DP), lambda ph, i: (i, 0)),
        out_shape=jax.ShapeDtypeStruct((N, DP), jnp.float32),
        scratch_shapes=[
            pltpu.VMEM((1, D), jnp.float32),
            pltpu.VMEM((1, D), jnp.float32),
            pltpu.VMEM((N, D), jnp.float32),
        ],
    )(p, x, Wl, Wr)


def _knn_kernel(q_ref, c_ref, sq_ref, idx_ref, vals, inds):
    ch = pl.program_id(1)

    @pl.when(ch == 0)
    def _init():
        vals[...] = jnp.full_like(vals[...], jnp.inf)
        inds[...] = jnp.zeros_like(inds[...])

    q = q_ref[...]
    c = c_ref[0]
    d2 = (q[:, 0:1] - c[0:1, :]) ** 2
    d2 = d2 + (q[:, 1:2] - c[1:2, :]) ** 2
    d2 = d2 + (q[:, 2:3] - c[2:3, :]) ** 2

    base = ch * KNN_CB
    BIG = jnp.int32(1 << 30)
    lane = lax.broadcasted_iota(jnp.int32, d2.shape, 1)
    cvs, cis = [], []
    work = d2
    for _ in range(3):
        m = jnp.min(work, axis=1, keepdims=True)
        am = jnp.min(jnp.where(work == m, lane, BIG), axis=1, keepdims=True)
        cvs.append(m)
        cis.append(am + base)
        work = jnp.where(lane == am, jnp.inf, work)

    catv = jnp.concatenate([vals[:, 0:3]] + cvs, axis=1)
    cati = jnp.concatenate([inds[:, 0:3]] + cis, axis=1)
    lane6 = lax.broadcasted_iota(jnp.int32, catv.shape, 1)
    nvs, nis = [], []
    for _ in range(3):
        m = jnp.min(catv, axis=1, keepdims=True)
        am = jnp.min(jnp.where(catv == m, lane6, BIG), axis=1, keepdims=True)
        sel = lane6 == am
        nvs.append(m)
        nis.append(jnp.sum(jnp.where(sel, cati, 0), axis=1, keepdims=True))
        catv = jnp.where(sel, jnp.inf, catv)
    vals[:, 0:3] = jnp.concatenate(nvs, axis=1)
    inds[:, 0:3] = jnp.concatenate(nis, axis=1)

    @pl.when(ch == KNN_NCH - 1)
    def _flush():
        sq_ref[:, 0:3] = vals[:, 0:3]
        sq_ref[:, 3:] = jnp.ones((sq_ref.shape[0], 5), jnp.float32)
        idx_ref[:, 0:3] = inds[:, 0:3]
        idx_ref[:, 3:] = jnp.zeros((idx_ref.shape[0], 5), jnp.int32)


def _knn_top3(h_pos, l_posT_ch):
    nqb = N // KNN_QB
    return pl.pallas_call(
        _knn_kernel,
        grid=(nqb, KNN_NCH),
        in_specs=[
            pl.BlockSpec((KNN_QB, 3), lambda qb, ch: (qb, 0)),
            pl.BlockSpec((1, 3, KNN_CB), lambda qb, ch: (ch, 0, 0)),
        ],
        out_specs=[
            pl.BlockSpec((KNN_QB, 8), lambda qb, ch: (qb, 0)),
            pl.BlockSpec((KNN_QB, 8), lambda qb, ch: (qb, 0)),
        ],
        out_shape=[
            jax.ShapeDtypeStruct((N, 8), jnp.float32),
            jax.ShapeDtypeStruct((N, 8), jnp.int32),
        ],
        scratch_shapes=[
            pltpu.VMEM((KNN_QB, 8), jnp.float32),
            pltpu.VMEM((KNN_QB, 8), jnp.int32),
        ],
    )(h_pos, l_posT_ch)


def _interp_kernel(rows_ref, sq_ref, out_ref):
    w = 1.0 / jnp.clip(sq_ref[:, 0:3], 1e-16, None)
    num = w[:, 0:1] * rows_ref[:, 0, :D]
    num = num + w[:, 1:2] * rows_ref[:, 1, :D]
    num = num + w[:, 2:3] * rows_ref[:, 2, :D]
    den = w[:, 0:1] + w[:, 1:2]
    den = den + w[:, 2:3]
    out_ref[:, :D] = num / den
    out_ref[:, D:] = jnp.ones((out_ref.shape[0], DP - D), jnp.float32)


def _interp(rows, sq):
    BM = 1000
    nb = N // BM
    return pl.pallas_call(
        _interp_kernel,
        grid=(nb,),
        in_specs=[
            pl.BlockSpec((BM, 3, DP), lambda i: (i, 0, 0)),
            pl.BlockSpec((BM, 8), lambda i: (i, 0)),
        ],
        out_specs=pl.BlockSpec((BM, DP), lambda i: (i, 0)),
        out_shape=jax.ShapeDtypeStruct((N, DP), jnp.float32),
    )(rows, sq)


def kernel(l_pos1, l_y1, l_e1, h_pos1, h_e1, Wenc1, benc1, Wenc2, benc2,
           Wl1, bl1, Wr1, Wl2, bl2, Wr2):
    sc_scatter = _build_sc_scatter()
    sc_gather = _build_sc_gather()

    zeros = jnp.zeros((ZSTRIPE, DP), jnp.float32)

    def edge_parts(e):
        src = e[0].astype(jnp.int32)
        dst = e[1].astype(jnp.int32)
        pad = NCHUNK * 128 - E
        src = jnp.concatenate([src, jnp.zeros((pad,), jnp.int32)])
        dst = jnp.concatenate([dst, jnp.full((pad,), N, jnp.int32)])
        return src.reshape(NCHUNK, 128), dst.reshape(NCHUNK, 128)

    l_src, l_dst = edge_parts(l_e1)
    h_src, h_dst = edge_parts(h_e1)

    xin = jnp.concatenate([l_y1, l_pos1], axis=-1)
    x = _encoder(xin, Wenc1, benc1, Wenc2, benc2)

    for i in range(2):
        p = sc_scatter(x, l_src, l_dst, zeros)
        x = _sage_dense(p, x, Wl1[i], Wr1[i])

    l_posT_ch = l_pos1.T.reshape(3, KNN_NCH, KNN_CB).transpose(1, 0, 2)
    sq, idx = _knn_top3(h_pos1, l_posT_ch)
    idx_flat = idx[:, 0:3].reshape(-1)
    idx_flat = jnp.concatenate(
        [idx_flat, jnp.zeros((GIDX - 3 * N,), jnp.int32)]).reshape(GNCH, 128)
    rows = sc_gather(x, idx_flat).reshape(GIDX // 3, 3, DP)
    x = _interp(rows, sq)

    for i in range(2):
        p = sc_scatter(x, h_src, h_dst, zeros)
        x = _sage_dense(p, x, Wl2[i], Wr2[i])

    return x[:, :D]


# speedup vs baseline: 1.6802x; 1.0414x over previous
import functools

import jax
import jax.numpy as jnp
from jax import lax
from jax.experimental import pallas as pl
from jax.experimental.pallas import tpu as pltpu
from jax.experimental.pallas import tpu_sc as plsc

N = 10000
D = 128
DP = 144
E = 320000
NC, NS = 2, 16
NW = NC * NS
ECH0 = 118
ECH1 = 40
NCHUNK = NS * (ECH0 + ECH1)
AGG_ROWS = 10112
ZSTRIPE = 632

KNN_CB = 2000
KNN_NCH = 5
KNN_QB = 400

GCH0 = 14
GCH1 = 1
GNCH = NS * (GCH0 + GCH1)
GIDX = GNCH * 128


def _build_sc_scatter():
    mesh = plsc.VectorSubcoreMesh(core_axis_name="c", subcore_axis_name="s")

    @functools.partial(
        pl.kernel,
        out_type=jax.ShapeDtypeStruct((NC, N, DP), jnp.float32),
        mesh=mesh,
        scratch_types=[
            pltpu.VMEM((2, 128), jnp.int32),
            pltpu.VMEM((2, 128), jnp.int32),
            pltpu.VMEM((2, 128, DP), jnp.float32),
            pltpu.VMEM_SHARED((AGG_ROWS, DP), jnp.float32),
            pltpu.SemaphoreType.DMA,
            pltpu.SemaphoreType.DMA,
            pltpu.SemaphoreType.DMA,
            pltpu.SemaphoreType.DMA,
        ],
        compiler_params=pltpu.CompilerParams(use_tc_tiling_on_sc=False),
    )
    def sc_scatter(table, src_r, dst_r, zeros, out, src_v, dst_v, rows_v,
                   agg_sh, sem_i, sem_g0, sem_g1, sem_s):
        c = lax.axis_index("c")
        s = lax.axis_index("s")
        pltpu.sync_copy(zeros, agg_sh.at[pl.ds(s * ZSTRIPE, ZSTRIPE)])
        plsc.subcore_barrier()

        sem_g = [sem_g0, sem_g1]

        def run(base, nch):
            @pl.loop(0, nch, step=2)
            def _group(j):
                ids = []
                for b in range(2):
                    ids.append(pltpu.async_copy(src_r.at[base + j + b],
                                                src_v.at[b], sem_i))
                    ids.append(pltpu.async_copy(dst_r.at[base + j + b],
                                                dst_v.at[b], sem_i))
                for d in ids:
                    d.wait()
                gds = [pltpu.async_copy(table.at[src_v.at[b]],
                                        rows_v.at[b], sem_g[b])
                       for b in range(2)]
                sds = []
                for b in range(2):
                    gds[b].wait()
                    sds.append(pltpu.async_copy(rows_v.at[b],
                                                agg_sh.at[dst_v.at[b]],
                                                sem_s, add=True))
                for d in sds:
                    d.wait()

        @pl.when(c == 0)
        def _core0():
            run(s * ECH0, ECH0)

        @pl.when(c == 1)
        def _core1():
            run(NS * ECH0 + s * ECH1, ECH1)

        plsc.subcore_barrier()

        @pl.when(s < NS - 1)
        def _copy_full():
            pltpu.sync_copy(agg_sh.at[pl.ds(s * ZSTRIPE, ZSTRIPE)],
                            out.at[c, pl.ds(s * ZSTRIPE, ZSTRIPE)])

        @pl.when(s == NS - 1)
        def _copy_tail():
            pltpu.sync_copy(agg_sh.at[pl.ds((NS - 1) * ZSTRIPE, N - (NS - 1) * ZSTRIPE)],
                            out.at[c, pl.ds((NS - 1) * ZSTRIPE, N - (NS - 1) * ZSTRIPE)])

    return sc_scatter


def _build_sc_gather():
    mesh = plsc.VectorSubcoreMesh(core_axis_name="c", subcore_axis_name="s")

    @functools.partial(
        pl.kernel,
        out_type=jax.ShapeDtypeStruct((GIDX, DP), jnp.float32),
        mesh=mesh,
        scratch_types=[
            pltpu.VMEM((GCH0, 128), jnp.int32),
            pltpu.VMEM((2, 128, DP), jnp.float32),
            pltpu.SemaphoreType.DMA,
            pltpu.SemaphoreType.DMA,
            pltpu.SemaphoreType.DMA,
        ],
        compiler_params=pltpu.CompilerParams(use_tc_tiling_on_sc=False),
    )
    def sc_gather(table, idx_r, out, idx_v, rows_v, sem_g0, sem_g1, sem_s):
        c = lax.axis_index("c")
        s = lax.axis_index("s")
        sem_g = [sem_g0, sem_g1]

        cbase = jnp.where(c == 0, s * GCH0, NS * GCH0 + s * GCH1)

        @pl.when(c == 0)
        def _core0():
            pltpu.sync_copy(idx_r.at[pl.ds(cbase, GCH0)], idx_v)

            @pl.loop(0, GCH0, step=2)
            def _group(j):
                gds = [pltpu.async_copy(table.at[idx_v.at[j + b]],
                                        rows_v.at[b], sem_g[b])
                       for b in range(2)]
                sds = []
                for b in range(2):
                    gds[b].wait()
                    sds.append(pltpu.async_copy(
                        rows_v.at[b],
                        out.at[pl.ds((cbase + j + b) * 128, 128)], sem_s))
                for d in sds:
                    d.wait()

        @pl.when(c == 1)
        def _core1():
            pltpu.sync_copy(idx_r.at[pl.ds(cbase, GCH1)],
                            idx_v.at[pl.ds(0, GCH1)])
            pltpu.async_copy(table.at[idx_v.at[0]], rows_v.at[0],
                             sem_g0).wait()
            pltpu.sync_copy(rows_v.at[0], out.at[pl.ds(cbase * 128, 128)])

    return sc_gather


def _enc_kernel(x_ref, w1_ref, b1_ref, w2_ref, b2_ref, out_ref):
    x = x_ref[...]
    h = lax.dot_general(x, w1_ref[...], (((1,), (1,)), ((), ())),
                        preferred_element_type=jnp.float32) + b1_ref[...]
    h = jnp.maximum(h, 0.0)
    y = lax.dot_general(h, w2_ref[...], (((1,), (1,)), ((), ())),
                        preferred_element_type=jnp.float32) + b2_ref[...]
    out_ref[:, :D] = y
    out_ref[:, D:] = jnp.ones((out_ref.shape[0], DP - D), jnp.float32)


def _encoder(xin, W1, b1, W2, b2):
    BM = 1000
    nb = N // BM
    return pl.pallas_call(
        _enc_kernel,
        grid=(nb,),
        in_specs=[
            pl.BlockSpec((BM, xin.shape[1]), lambda i: (i, 0)),
            pl.BlockSpec(W1.shape, lambda i: (0, 0)),
            pl.BlockSpec((1, D), lambda i: (0, 0)),
            pl.BlockSpec(W2.shape, lambda i: (0, 0)),
            pl.BlockSpec((1, D), lambda i: (0, 0)),
        ],
        out_specs=pl.BlockSpec((BM, DP), lambda i: (i, 0)),
        out_shape=jax.ShapeDtypeStruct((N, DP), jnp.float32),
    )(xin, W1, b1.reshape(1, D), W2, b2.reshape(1, D))


def _sage_dense_kernel(p_ref, x_ref, wl_ref, wr_ref, out_ref,
                       acc_agg, acc_y, yraw, *, bm):
    ph = pl.program_id(0)
    i = pl.program_id(1)

    @pl.when(jnp.logical_and(ph == 0, i == 0))
    def _init():
        acc_agg[...] = jnp.zeros_like(acc_agg)
        acc_y[...] = jnp.zeros_like(acc_y)

    def _agg_block():
        feat = p_ref[0, :, :D] + p_ref[1, :, :D]
        cnt = p_ref[0, :, D:D + 1] + p_ref[1, :, D:D + 1]
        return feat / jnp.clip(cnt, 1.0, None)

    @pl.when(ph == 0)
    def _phase0():
        acc_agg[...] += jnp.sum(_agg_block(), axis=0, keepdims=True)

    @pl.when(ph == 1)
    def _phase1():
        agg = _agg_block() - acc_agg[...] / float(N)
        xf = x_ref[:, :D]
        t = lax.dot_general(agg, wl_ref[...], (((1,), (1,)), ((), ())),
                            preferred_element_type=jnp.float32)
        t = t + lax.dot_general(xf, wr_ref[...], (((1,), (1,)), ((), ())),
                                preferred_element_type=jnp.float32)
        yr = xf + jnp.maximum(t, 0.0)
        acc_y[...] += jnp.sum(yr, axis=0, keepdims=True)
        yraw[pl.ds(i * bm, bm), :] = yr

    @pl.when(ph == 2)
    def _phase2():
        out_ref[:, :D] = yraw[pl.ds(i * bm, bm), :] - acc_y[...] / float(N)
        out_ref[:, D:] = jnp.ones((bm, DP - D), jnp.float32)


def _sage_dense(p, x, Wl, Wr):
    BM = 1000
    nb = N // BM
    return pl.pallas_call(
        functools.partial(_sage_dense_kernel, bm=BM),
        grid=(3, nb),
        in_specs=[
            pl.BlockSpec((2, BM, DP), lambda ph, i: (0, i, 0)),
            pl.BlockSpec((BM, DP), lambda ph, i: (i, 0)),
            pl.BlockSpec((D, D), lambda ph, i: (0, 0)),
            pl.BlockSpec((D, D), lambda ph, i: (0, 0)),
        ],
        out_specs=pl.BlockSpec((BM, DP), lambda ph, i: (i, 0)),
        out_shape=jax.ShapeDtypeStruct((N, DP), jnp.float32),
        scratch_shapes=[
            pltpu.VMEM((1, D), jnp.float32),
            pltpu.VMEM((1, D), jnp.float32),
            pltpu.VMEM((N, D), jnp.float32),
        ],
    )(p, x, Wl, Wr)


def _knn_kernel(q_ref, c_ref, sq_ref, idx_ref, vals, inds):
    ch = pl.program_id(1)

    @pl.when(ch == 0)
    def _init():
        vals[...] = jnp.full_like(vals[...], jnp.inf)
        inds[...] = jnp.zeros_like(inds[...])

    q = q_ref[...]
    c = c_ref[0]
    d2 = (q[:, 0:1] - c[0:1, :]) ** 2
    d2 = d2 + (q[:, 1:2] - c[1:2, :]) ** 2
    d2 = d2 + (q[:, 2:3] - c[2:3, :]) ** 2

    base = ch * KNN_CB
    BIG = jnp.int32(1 << 30)
    lane = lax.broadcasted_iota(jnp.int32, d2.shape, 1)
    cvs, cis = [], []
    work = d2
    for _ in range(3):
        m = jnp.min(work, axis=1, keepdims=True)
        am = jnp.min(jnp.where(work == m, lane, BIG), axis=1, keepdims=True)
        cvs.append(m)
        cis.append(am + base)
        work = jnp.where(lane == am, jnp.inf, work)

    cv3 = jnp.concatenate(cvs, axis=1)
    ci3 = jnp.concatenate(cis, axis=1)
    for k in range(KNN_NCH):
        @pl.when(ch == k)
        def _stash(k=k):
            vals[:, 3 * k:3 * k + 3] = cv3
            inds[:, 3 * k:3 * k + 3] = ci3

    @pl.when(ch == KNN_NCH - 1)
    def _flush():
        catv = vals[...]
        cati = inds[...]
        lane16 = lax.broadcasted_iota(jnp.int32, catv.shape, 1)
        nvs, nis = [], []
        for _ in range(3):
            m = jnp.min(catv, axis=1, keepdims=True)
            am = jnp.min(jnp.where(catv == m, lane16, BIG),
                         axis=1, keepdims=True)
            sel = lane16 == am
            nvs.append(m)
            nis.append(jnp.sum(jnp.where(sel, cati, 0), axis=1, keepdims=True))
            catv = jnp.where(sel, jnp.inf, catv)
        sq_ref[:, 0:3] = jnp.concatenate(nvs, axis=1)
        sq_ref[:, 3:] = jnp.ones((sq_ref.shape[0], 5), jnp.float32)
        idx_ref[:, 0:3] = jnp.concatenate(nis, axis=1)
        idx_ref[:, 3:] = jnp.zeros((idx_ref.shape[0], 5), jnp.int32)


def _knn_top3(h_pos, l_posT_ch):
    nqb = N // KNN_QB
    return pl.pallas_call(
        _knn_kernel,
        grid=(nqb, KNN_NCH),
        in_specs=[
            pl.BlockSpec((KNN_QB, 3), lambda qb, ch: (qb, 0)),
            pl.BlockSpec((1, 3, KNN_CB), lambda qb, ch: (ch, 0, 0)),
        ],
        out_specs=[
            pl.BlockSpec((KNN_QB, 8), lambda qb, ch: (qb, 0)),
            pl.BlockSpec((KNN_QB, 8), lambda qb, ch: (qb, 0)),
        ],
        out_shape=[
            jax.ShapeDtypeStruct((N, 8), jnp.float32),
            jax.ShapeDtypeStruct((N, 8), jnp.int32),
        ],
        scratch_shapes=[
            pltpu.VMEM((KNN_QB, 16), jnp.float32),
            pltpu.VMEM((KNN_QB, 16), jnp.int32),
        ],
    )(h_pos, l_posT_ch)


def _interp_kernel(rows_ref, sq_ref, out_ref):
    w = 1.0 / jnp.clip(sq_ref[:, 0:3], 1e-16, None)
    num = w[:, 0:1] * rows_ref[:, 0, :D]
    num = num + w[:, 1:2] * rows_ref[:, 1, :D]
    num = num + w[:, 2:3] * rows_ref[:, 2, :D]
    den = w[:, 0:1] + w[:, 1:2]
    den = den + w[:, 2:3]
    out_ref[:, :D] = num / den
    out_ref[:, D:] = jnp.ones((out_ref.shape[0], DP - D), jnp.float32)


def _interp(rows, sq):
    BM = 1000
    nb = N // BM
    return pl.pallas_call(
        _interp_kernel,
        grid=(nb,),
        in_specs=[
            pl.BlockSpec((BM, 3, DP), lambda i: (i, 0, 0)),
            pl.BlockSpec((BM, 8), lambda i: (i, 0)),
        ],
        out_specs=pl.BlockSpec((BM, DP), lambda i: (i, 0)),
        out_shape=jax.ShapeDtypeStruct((N, DP), jnp.float32),
    )(rows, sq)


def kernel(l_pos1, l_y1, l_e1, h_pos1, h_e1, Wenc1, benc1, Wenc2, benc2,
           Wl1, bl1, Wr1, Wl2, bl2, Wr2):
    sc_scatter = _build_sc_scatter()
    sc_gather = _build_sc_gather()

    zeros = jnp.zeros((ZSTRIPE, DP), jnp.float32)

    def edge_parts(e):
        src = e[0].astype(jnp.int32)
        dst = e[1].astype(jnp.int32)
        pad = NCHUNK * 128 - E
        src = jnp.concatenate([src, jnp.zeros((pad,), jnp.int32)])
        dst = jnp.concatenate([dst, jnp.full((pad,), N, jnp.int32)])
        return src.reshape(NCHUNK, 128), dst.reshape(NCHUNK, 128)

    l_src, l_dst = edge_parts(l_e1)
    h_src, h_dst = edge_parts(h_e1)

    xin = jnp.concatenate([l_y1, l_pos1], axis=-1)
    x = _encoder(xin, Wenc1, benc1, Wenc2, benc2)

    for i in range(2):
        p = sc_scatter(x, l_src, l_dst, zeros)
        x = _sage_dense(p, x, Wl1[i], Wr1[i])

    l_posT_ch = l_pos1.T.reshape(3, KNN_NCH, KNN_CB).transpose(1, 0, 2)
    sq, idx = _knn_top3(h_pos1, l_posT_ch)
    idx_flat = idx[:, 0:3].reshape(-1)
    idx_flat = jnp.concatenate(
        [idx_flat, jnp.zeros((GIDX - 3 * N,), jnp.int32)]).reshape(GNCH, 128)
    rows = sc_gather(x, idx_flat).reshape(GIDX // 3, 3, DP)
    x = _interp(rows, sq)

    for i in range(2):
        p = sc_scatter(x, h_src, h_dst, zeros)
        x = _sage_dense(p, x, Wl2[i], Wr2[i])

    return x[:, :D]
